# baseline (speedup 1.0000x reference)
import functools

import jax
import jax.numpy as jnp
from jax import lax
from jax.experimental import pallas as pl
from jax.experimental.pallas import tpu as pltpu
from jax.experimental.pallas import tpu_sc as plsc

_NSEL = 64
_CHUNK = 1024
_SCH = 2
_TCH = 8 - _SCH


def _topk_indices(acc):
    n = acc.shape[-1]
    vrow = acc.reshape(1, n)
    vcols = lax.broadcast_in_dim(vrow, (n, n), (0, 1))
    vcol1 = jnp.transpose(vrow, (1, 0))
    vrows = lax.broadcast_in_dim(vcol1, (n, n), (0, 1))
    jj = lax.broadcasted_iota(jnp.int32, (n, n), 0)
    ii = lax.broadcasted_iota(jnp.int32, (n, n), 1)
    beats = (vrows > vcols) | ((vrows == vcols) & (jj < ii))
    rank_row = jnp.sum(beats.astype(jnp.int32), axis=0, keepdims=True)
    rank_col = jnp.transpose(rank_row, (1, 0))
    rank_b = lax.broadcast_in_dim(rank_col, (n, _NSEL), (0, 1))
    rr = lax.broadcasted_iota(jnp.int32, (n, _NSEL), 1)
    ivals = lax.broadcasted_iota(jnp.int32, (n, _NSEL), 0)
    idxmat = jnp.where(rank_b == rr, ivals, 0)
    return jnp.sum(idxmat, axis=0, keepdims=True).reshape(1, 1, _NSEL)


def _reduce_tc(scores):
    B, H, N, NC = scores.shape
    nstream = _TCH * 2
    nsteps = N // _CHUNK // 2

    def body(*refs):
        s_refs, (acc_ref,) = refs[:nstream], refs[nstream:]
        c = pl.program_id(1)
        partial = s_refs[0][...].sum(axis=2)
        for j in range(1, nstream):
            partial = partial + s_refs[j][...].sum(axis=2)

        @pl.when(c == 0)
        def _():
            acc_ref[...] = partial

        @pl.when(c != 0)
        def _():
            acc_ref[...] = acc_ref[...] + partial

    def mk_spec(j):
        h, p = j // 2, j % 2
        return pl.BlockSpec(
            (1, 1, _CHUNK, NC), lambda b, c, h=h, p=p: (b, h, c * 2 + p, 0))

    return pl.pallas_call(
        body,
        grid=(B, nsteps),
        in_specs=[mk_spec(j) for j in range(nstream)],
        out_specs=[pl.BlockSpec((1, 1, NC), lambda b, c: (b, 0, 0))],
        out_shape=[jax.ShapeDtypeStruct((B, 1, NC), jnp.float32)],
    )(*([scores] * nstream))[0]


def _reduce_sc(scores2):
    mesh = plsc.VectorSubcoreMesh(core_axis_name="c", subcore_axis_name="s")
    nchunks = 16
    rows_per_chunk = 64

    @functools.partial(
        pl.kernel,
        mesh=mesh,
        out_type=[jax.ShapeDtypeStruct((32, 256), jnp.float32)],
        scratch_types=[
            pltpu.VMEM((rows_per_chunk, 256), jnp.float32),
            pltpu.VMEM((rows_per_chunk, 256), jnp.float32),
            pltpu.VMEM((256,), jnp.float32),
            pltpu.SemaphoreType.DMA,
            pltpu.SemaphoreType.DMA,
        ],
    )
    def sck(s_h, out_h, buf0, buf1, accv, sem0, sem1):
        w = lax.axis_index("s") * 2 + lax.axis_index("c")
        b = w // 8
        rem = w % 8
        h = _TCH + rem // 4
        sl = rem % 4
        base_row = (b * 8 + h) * 4096 + sl * 1024

        bufs = (buf0, buf1)
        sems = (sem0, sem1)

        def start(c):
            return pltpu.async_copy(
                s_h.at[pl.ds(base_row + c * rows_per_chunk, rows_per_chunk), :],
                bufs[c % 2], sems[c % 2])

        handles = {0: start(0)}
        accs = tuple(jnp.zeros((16,), jnp.float32) for _ in range(16))
        for c in range(nchunks):
            if c + 1 < nchunks:
                handles[c + 1] = start(c + 1)
            handles[c].wait()
            buf = bufs[c % 2]

            def rowbody(r, a):
                return tuple(
                    a[g] + buf[r, pl.ds(g * 16, 16)] for g in range(16))

            accs = lax.fori_loop(0, rows_per_chunk, rowbody, accs)

        for g in range(16):
            accv[pl.ds(g * 16, 16)] = accs[g]
        pltpu.sync_copy(accv, out_h.at[w])

    return sck(scores2)[0]


def _combine_topk(acc_tc, acc_sc):
    B = acc_tc.shape[0]

    def body(a_ref, s_ref, idx_ref):
        tot = a_ref[...] + jnp.sum(s_ref[...], axis=1, keepdims=True)
        idx_ref[...] = _topk_indices(tot)

    idx = pl.pallas_call(
        body,
        grid=(B,),
        in_specs=[
            pl.BlockSpec((1, 1, 256), lambda b: (b, 0, 0)),
            pl.BlockSpec((1, 8, 256), lambda b: (b, 0, 0)),
        ],
        out_specs=[pl.BlockSpec((1, 1, _NSEL), lambda b: (b, 0, 0))],
        out_shape=[jax.ShapeDtypeStruct((B, 1, _NSEL), jnp.int32)],
    )(acc_tc, acc_sc)[0]
    return idx.reshape(B, _NSEL)


def _gather_sc(kr, vr, idx):
    mesh = plsc.VectorSubcoreMesh(core_axis_name="c", subcore_axis_name="s")

    @functools.partial(
        pl.kernel,
        mesh=mesh,
        out_type=[
            jax.ShapeDtypeStruct((1024, 1024), jnp.float32),
            jax.ShapeDtypeStruct((1024, 1024), jnp.float32),
        ],
        scratch_types=[
            pltpu.VMEM((16,), jnp.int32),
            pltpu.VMEM((64,), jnp.int32),
            pltpu.VMEM((64, 1024), jnp.float32),
            pltpu.SemaphoreType.DMA,
        ],
    )
    def sck(kr_h, vr_h, idx_h, gk_h, gv_h, idxv, rows, inb, sem):
        wid = lax.axis_index("s") * 2 + lax.axis_index("c")
        tensor = wid // 16
        w2 = wid % 16
        b = w2 // 4
        s0 = (w2 % 4) * 16

        pltpu.sync_copy(idx_h.at[b, pl.ds(s0, 16)], idxv)
        ivec = idxv[...]
        base = b * 1024 + lax.div(ivec, 16) * 64 + lax.rem(ivec, 16)
        for r in range(4):
            rows[pl.ds(r * 16, 16)] = base + r * 16

        @pl.when(tensor == 0)
        def _():
            pltpu.async_copy(kr_h.at[rows], inb, sem).wait()
            pltpu.sync_copy(inb, gk_h.at[pl.ds(w2 * 64, 64), :])

        @pl.when(tensor == 1)
        def _():
            pltpu.async_copy(vr_h.at[rows], inb, sem).wait()
            pltpu.sync_copy(inb, gv_h.at[pl.ds(w2 * 64, 64), :])

    return sck(kr, vr, idx)


def _scramble_tc(gk, gv):
    gk6 = gk.reshape(16, 4, 16, 4, 256)
    gv6 = gv.reshape(16, 4, 16, 4, 256)

    def body(k_ref, v_ref, ok_ref, ov_ref):
        for tl in range(16):
            xk = k_ref[0, :, tl, :, :].reshape(16, 256)
            ok_ref[tl] = jnp.transpose(xk, (1, 0))
            xv = v_ref[0, :, tl, :, :].reshape(16, 256)
            ov_ref[tl] = jnp.transpose(xv, (1, 0))

    in_spec = pl.BlockSpec((1, 4, 16, 4, 256), lambda w: (w, 0, 0, 0, 0))
    out_spec = pl.BlockSpec((16, 256, 16), lambda w: (w, 0, 0))
    tk, tv = pl.pallas_call(
        body,
        grid=(16,),
        in_specs=[in_spec, in_spec],
        out_specs=[out_spec, out_spec],
        out_shape=[
            jax.ShapeDtypeStruct((256, 256, 16), jnp.float32),
            jax.ShapeDtypeStruct((256, 256, 16), jnp.float32),
        ],
    )(gk6, gv6)
    return tk, tv


def kernel(q, k, v, attn_scores_cmp, spatial_size):
    del q, spatial_size
    B, H, N, NC = attn_scores_cmp.shape
    acc_sc = _reduce_sc(attn_scores_cmp.reshape(B * H * N, NC))
    acc_tc = _reduce_tc(attn_scores_cmp)
    indices = _combine_topk(acc_tc, acc_sc.reshape(B, 8, 256))
    kr = k.reshape(B * 1024, 1024)
    vr = v.reshape(B * 1024, 1024)
    gk, gv = _gather_sc(kr, vr, indices)
    tk, tv = _scramble_tc(gk, gv)
    k_slc = tk.reshape(B, _NSEL * 16, 256)
    v_slc = tv.reshape(B, _NSEL * 16, 256)
    return (k_slc, v_slc, indices)


# speedup vs baseline: 1.0648x; 1.0648x over previous
import functools

import jax
import jax.numpy as jnp
from jax import lax
from jax.experimental import pallas as pl
from jax.experimental.pallas import tpu as pltpu
from jax.experimental.pallas import tpu_sc as plsc

_NSEL = 64
_CHUNK = 1024
_SCH = 8
_TCH = 8 - _SCH


def _topk_indices(acc):
    n = acc.shape[-1]
    vrow = acc.reshape(1, n)
    vcols = lax.broadcast_in_dim(vrow, (n, n), (0, 1))
    vcol1 = jnp.transpose(vrow, (1, 0))
    vrows = lax.broadcast_in_dim(vcol1, (n, n), (0, 1))
    jj = lax.broadcasted_iota(jnp.int32, (n, n), 0)
    ii = lax.broadcasted_iota(jnp.int32, (n, n), 1)
    beats = (vrows > vcols) | ((vrows == vcols) & (jj < ii))
    rank_row = jnp.sum(beats.astype(jnp.int32), axis=0, keepdims=True)
    rank_col = jnp.transpose(rank_row, (1, 0))
    rank_b = lax.broadcast_in_dim(rank_col, (n, _NSEL), (0, 1))
    rr = lax.broadcasted_iota(jnp.int32, (n, _NSEL), 1)
    ivals = lax.broadcasted_iota(jnp.int32, (n, _NSEL), 0)
    idxmat = jnp.where(rank_b == rr, ivals, 0)
    return jnp.sum(idxmat, axis=0, keepdims=True).reshape(1, 1, _NSEL)


def _reduce_sc(scores2):
    mesh = plsc.VectorSubcoreMesh(core_axis_name="c", subcore_axis_name="s")
    rpc = 64
    nchunks = 4096 // rpc
    nbuf = 4

    @functools.partial(
        pl.kernel,
        mesh=mesh,
        out_type=[jax.ShapeDtypeStruct((32, 256), jnp.float32)],
        scratch_types=(
            [pltpu.VMEM((rpc, 256), jnp.float32) for _ in range(nbuf)]
            + [pltpu.VMEM((256,), jnp.float32)]
            + [pltpu.SemaphoreType.DMA for _ in range(nbuf)]
        ),
    )
    def sck(s_h, out_h, *scr):
        bufs, accv, sems = scr[:nbuf], scr[nbuf], scr[nbuf + 1:]
        w = lax.axis_index("s") * 2 + lax.axis_index("c")
        base_row = w * 4096

        def start(c, j):
            return pltpu.async_copy(
                s_h.at[pl.ds(base_row + c * rpc, rpc), :], bufs[j], sems[j])

        for j in range(nbuf):
            start(j, j)

        accs = tuple(jnp.zeros((16,), jnp.float32) for _ in range(16))

        def group(i, accs):
            c0 = i * nbuf
            for j in range(nbuf):
                pltpu.make_async_copy(
                    s_h.at[pl.ds(base_row, rpc), :], bufs[j], sems[j]).wait()
                buf = bufs[j]

                def rowbody(r, a, buf=buf):
                    r0 = r * 2
                    a = tuple(
                        a[g] + buf[r0, pl.ds(g * 16, 16)] for g in range(16))
                    return tuple(
                        a[g] + buf[r0 + 1, pl.ds(g * 16, 16)]
                        for g in range(16))

                accs = lax.fori_loop(0, rpc // 2, rowbody, accs)

                @pl.when(c0 + nbuf + j < nchunks)
                def _(c0=c0, j=j):
                    start(c0 + nbuf + j, j)
            return accs

        accs = lax.fori_loop(0, nchunks // nbuf, group, accs)

        for g in range(16):
            accv[pl.ds(g * 16, 16)] = accs[g]
        pltpu.sync_copy(accv, out_h.at[w])

    return sck(scores2)[0]


def _combine_topk(acc_sc):
    B = acc_sc.shape[0]

    def body(s_ref, idx_ref):
        tot = jnp.sum(s_ref[...], axis=1, keepdims=True)
        idx_ref[...] = _topk_indices(tot)

    idx = pl.pallas_call(
        body,
        grid=(B,),
        in_specs=[pl.BlockSpec((1, 8, 256), lambda b: (b, 0, 0))],
        out_specs=[pl.BlockSpec((1, 1, _NSEL), lambda b: (b, 0, 0))],
        out_shape=[jax.ShapeDtypeStruct((B, 1, _NSEL), jnp.int32)],
    )(acc_sc)[0]
    return idx.reshape(B, _NSEL)


def _gather_sc(kr, vr, idx):
    mesh = plsc.VectorSubcoreMesh(core_axis_name="c", subcore_axis_name="s")

    @functools.partial(
        pl.kernel,
        mesh=mesh,
        out_type=[
            jax.ShapeDtypeStruct((1024, 1024), jnp.float32),
            jax.ShapeDtypeStruct((1024, 1024), jnp.float32),
        ],
        scratch_types=[
            pltpu.VMEM((16,), jnp.int32),
            pltpu.VMEM((64,), jnp.int32),
            pltpu.VMEM((64, 1024), jnp.float32),
            pltpu.SemaphoreType.DMA,
        ],
    )
    def sck(kr_h, vr_h, idx_h, gk_h, gv_h, idxv, rows, inb, sem):
        wid = lax.axis_index("s") * 2 + lax.axis_index("c")
        tensor = wid // 16
        w2 = wid % 16
        b = w2 // 4
        s0 = (w2 % 4) * 16

        pltpu.sync_copy(idx_h.at[b, pl.ds(s0, 16)], idxv)
        ivec = idxv[...]
        base = b * 1024 + lax.div(ivec, 16) * 64 + lax.rem(ivec, 16)
        for r in range(4):
            rows[pl.ds(r * 16, 16)] = base + r * 16

        @pl.when(tensor == 0)
        def _():
            pltpu.async_copy(kr_h.at[rows], inb, sem).wait()
            pltpu.sync_copy(inb, gk_h.at[pl.ds(w2 * 64, 64), :])

        @pl.when(tensor == 1)
        def _():
            pltpu.async_copy(vr_h.at[rows], inb, sem).wait()
            pltpu.sync_copy(inb, gv_h.at[pl.ds(w2 * 64, 64), :])

    return sck(kr, vr, idx)


def _scramble_tc(gk, gv):
    gk6 = gk.reshape(16, 4, 16, 4, 256)
    gv6 = gv.reshape(16, 4, 16, 4, 256)

    def body(k_ref, v_ref, ok_ref, ov_ref):
        for tl in range(16):
            xk = k_ref[0, :, tl, :, :].reshape(16, 256)
            ok_ref[tl] = jnp.transpose(xk, (1, 0))
            xv = v_ref[0, :, tl, :, :].reshape(16, 256)
            ov_ref[tl] = jnp.transpose(xv, (1, 0))

    in_spec = pl.BlockSpec((1, 4, 16, 4, 256), lambda w: (w, 0, 0, 0, 0))
    out_spec = pl.BlockSpec((16, 256, 16), lambda w: (w, 0, 0))
    tk, tv = pl.pallas_call(
        body,
        grid=(16,),
        in_specs=[in_spec, in_spec],
        out_specs=[out_spec, out_spec],
        out_shape=[
            jax.ShapeDtypeStruct((256, 256, 16), jnp.float32),
            jax.ShapeDtypeStruct((256, 256, 16), jnp.float32),
        ],
    )(gk6, gv6)
    return tk, tv


def kernel(q, k, v, attn_scores_cmp, spatial_size):
    del q, spatial_size
    B, H, N, NC = attn_scores_cmp.shape
    acc_sc = _reduce_sc(attn_scores_cmp.reshape(B * H * N, NC))
    indices = _combine_topk(acc_sc.reshape(B, 8, 256))
    kr = k.reshape(B * 1024, 1024)
    vr = v.reshape(B * 1024, 1024)
    gk, gv = _gather_sc(kr, vr, indices)
    tk, tv = _scramble_tc(gk, gv)
    k_slc = tk.reshape(B, _NSEL * 16, 256)
    v_slc = tv.reshape(B, _NSEL * 16, 256)
    return (k_slc, v_slc, indices)
